# baseline (speedup 1.0000x reference)
import functools

import jax
import jax.numpy as jnp
from jax import lax
from jax.experimental import pallas as pl
from jax.experimental.pallas import tpu as pltpu
from jax.experimental.pallas import tpu_sc as plsc

N = 10000
H = 16
B = 16
E = 160000
NB = 1024
C = 1600
NCHUNK = E // C


def _sc_body(cal_hbm, src_hbm, tgt_hbm, wz_hbm, wr_hbm, wh_hbm, out_hbm,
             cal_v, src_v, tgt_v, w_v, acc_v, sem0, sem1):
    cid = lax.axis_index("c")
    sid = lax.axis_index("s")
    h0 = cid * 8
    pltpu.sync_copy(cal_hbm.at[sid], cal_v)

    sems = (sem0, sem1)
    zero16 = jnp.zeros((16,), jnp.float32)

    for g, wt_hbm in enumerate((wz_hbm, wr_hbm, wh_hbm)):
        def start(slot, c, wt=wt_hbm):
            sem = sems[slot]
            pltpu.async_copy(src_hbm.at[pl.ds(c * C, C)], src_v.at[slot], sem)
            pltpu.async_copy(tgt_hbm.at[pl.ds(c * C, C)], tgt_v.at[slot], sem)
            pltpu.async_copy(wt.at[pl.ds(h0, 8), pl.ds(c * C, C)],
                             w_v.at[slot], sem)

        def wait(slot, c, wt=wt_hbm):
            sem = sems[slot]
            pltpu.make_async_copy(src_hbm.at[pl.ds(c * C, C)],
                                  src_v.at[slot], sem).wait()
            pltpu.make_async_copy(tgt_hbm.at[pl.ds(c * C, C)],
                                  tgt_v.at[slot], sem).wait()
            pltpu.make_async_copy(wt.at[pl.ds(h0, 8), pl.ds(c * C, C)],
                                  w_v.at[slot], sem).wait()

        def compute(slot):
            @plsc.parallel_loop(0, C // 16, unroll=4)
            def _(i):
                s16 = src_v[slot, pl.ds(i * 16, 16)]
                t16 = tgt_v[slot, pl.ds(i * 16, 16)]
                g16 = plsc.load_gather(cal_v, [s16])
                for j in range(8):
                    w16 = w_v[slot, j, pl.ds(i * 16, 16)]
                    idx = t16 + (j * N) if j else t16
                    plsc.addupdate_scatter(acc_v, [idx], g16 * w16)

        start(0, 0)

        @plsc.parallel_loop(0, (8 * N) // 16, unroll=8)
        def _(i):
            acc_v[pl.ds(i * 16, 16)] = zero16

        @pl.loop(0, NCHUNK, step=2)
        def _(c):
            start(1, c + 1)
            wait(0, c)
            compute(0)

            @pl.when(c + 2 < NCHUNK)
            def _():
                start(0, c + 2)

            wait(1, c + 1)
            compute(1)

        for j in range(8):
            pltpu.sync_copy(acc_v.at[pl.ds(j * N, N)], out_hbm.at[g, h0 + j, sid])


def _sc_sparse(calcium_t, src, tgt, wt_z, wt_r, wt_h):
    mesh = plsc.VectorSubcoreMesh(core_axis_name="c", subcore_axis_name="s")
    fn = pl.kernel(
        _sc_body,
        out_type=jax.ShapeDtypeStruct((3, H, B, N), jnp.float32),
        mesh=mesh,
        scratch_types=[
            pltpu.VMEM((N,), jnp.float32),
            pltpu.VMEM((2, C), jnp.int32),
            pltpu.VMEM((2, C), jnp.int32),
            pltpu.VMEM((2, 8, C), jnp.float32),
            pltpu.VMEM((8 * N,), jnp.float32),
            pltpu.SemaphoreType.DMA,
            pltpu.SemaphoreType.DMA,
        ],
        compiler_params=pltpu.CompilerParams(
            needs_layout_passes=False, use_tc_tiling_on_sc=False),
    )
    return fn(calcium_t, src, tgt, wt_z, wt_r, wt_h)


def _rec_body(hid_ref, uz_ref, ur_ref, bz_ref, br_ref, rz_ref, rr_ref):
    hid = hid_ref[...]
    uz = uz_ref[...]
    ur = ur_ref[...]
    rec_z = jnp.broadcast_to(bz_ref[...][:, None, :], (H, B, NB))
    rec_r = jnp.broadcast_to(br_ref[...][:, None, :], (H, B, NB))
    for h in range(H):
        hh = hid[h][None, :, :]
        rec_z = rec_z + hh * uz[h][:, None, :]
        rec_r = rec_r + hh * ur[h][:, None, :]
    rz_ref[...] = rec_z
    rr_ref[...] = rec_r


def _gate_body(hid_ref, uh_ref, iz_ref, ir_ref, ih_ref, rz_ref, rr_ref,
               bh_ref, w_ref, hnew_ref, cal_ref):
    hid = hid_ref[...]
    uh = uh_ref[...]
    z = jax.nn.sigmoid(iz_ref[...] + rz_ref[...])
    r = jax.nn.sigmoid(ir_ref[...] + rr_ref[...])

    rh = r * hid
    rec_h = jnp.zeros((H, B, NB), jnp.float32)
    for h in range(H):
        rec_h = rec_h + rh[h][None, :, :] * uh[h][:, None, :]
    h_t = jnp.tanh(ih_ref[...] + rec_h + bh_ref[...][:, None, :])

    hnew = (1.0 - z) * hid + z * h_t
    hnew_ref[...] = hnew

    w = w_ref[...][:, 0]
    cal_ref[...] = jax.nn.relu(jnp.sum(hnew * w[:, None, None], axis=0))


_HBN = pl.BlockSpec((H, B, NB), lambda j: (0, 0, j))
_HN = pl.BlockSpec((H, NB), lambda j: (0, j))


def _rec_call(hid_t, u_z, u_r, b_z, b_r):
    return pl.pallas_call(
        _rec_body,
        grid=(pl.cdiv(N, NB),),
        in_specs=[_HBN, _HBN, _HBN, _HN, _HN],
        out_specs=[_HBN, _HBN],
        out_shape=[
            jax.ShapeDtypeStruct((H, B, N), jnp.float32),
            jax.ShapeDtypeStruct((H, B, N), jnp.float32),
        ],
    )(hid_t, u_z, u_r, b_z, b_r)


def _gate_call(hid_t, u_h, i_z, i_r, i_h, r_z, r_r, b_h, w2d):
    wspec = pl.BlockSpec((H, 128), lambda j: (0, 0))
    return pl.pallas_call(
        _gate_body,
        grid=(pl.cdiv(N, NB),),
        in_specs=[_HBN, _HBN, _HBN, _HBN, _HBN, _HBN, _HBN, _HN, wspec],
        out_specs=[_HBN, pl.BlockSpec((B, NB), lambda j: (0, j))],
        out_shape=[
            jax.ShapeDtypeStruct((H, B, N), jnp.float32),
            jax.ShapeDtypeStruct((B, N), jnp.float32),
        ],
    )(hid_t, u_h, i_z, i_r, i_h, r_z, r_r, b_h, w2d)


def kernel(calcium_t, hidden, W_z_values, W_r_values, W_h_values,
           U_z, U_r, U_h, b_z, b_r, b_h, output_projection, src, tgt):
    hid_t = jnp.transpose(hidden, (2, 0, 1))
    u_z = jnp.transpose(U_z, (1, 2, 0))
    u_r = jnp.transpose(U_r, (1, 2, 0))
    u_h = jnp.transpose(U_h, (1, 2, 0))
    bz_t = b_z.T
    br_t = b_r.T
    bh_t = b_h.T
    w2d = jnp.broadcast_to(output_projection[:, None], (H, 128))
    wt_z = W_z_values.T
    wt_r = W_r_values.T
    wt_h = W_h_values.T

    inp = _sc_sparse(calcium_t, src, tgt, wt_z, wt_r, wt_h)

    r_z, r_r = _rec_call(hid_t, u_z, u_r, bz_t, br_t)

    hnew_t, calcium_t1 = _gate_call(hid_t, u_h, inp[0], inp[1], inp[2],
                                    r_z, r_r, bh_t, w2d)
    hidden_new = jnp.transpose(hnew_t, (1, 2, 0))
    return (calcium_t1, hidden_new)


# speedup vs baseline: 1.0054x; 1.0054x over previous
import functools

import jax
import jax.numpy as jnp
from jax import lax
from jax.experimental import pallas as pl
from jax.experimental.pallas import tpu as pltpu
from jax.experimental.pallas import tpu_sc as plsc

N = 10000
H = 16
B = 16
E = 160000
NB = 512
C = 1600
NCHUNK = E // C


def _sc_body(cal_hbm, src_hbm, tgt_hbm, wz_hbm, wr_hbm, wh_hbm, out_hbm,
             cal_v, src_v, tgt_v, w_v, acc_v, sem0, sem1):
    cid = lax.axis_index("c")
    sid = lax.axis_index("s")
    h0 = cid * 8
    pltpu.sync_copy(cal_hbm.at[sid], cal_v)

    sems = (sem0, sem1)
    zero16 = jnp.zeros((16,), jnp.float32)

    for g, wt_hbm in enumerate((wz_hbm, wr_hbm, wh_hbm)):
        def start(slot, c, wt=wt_hbm):
            sem = sems[slot]
            pltpu.async_copy(src_hbm.at[pl.ds(c * C, C)], src_v.at[slot], sem)
            pltpu.async_copy(tgt_hbm.at[pl.ds(c * C, C)], tgt_v.at[slot], sem)
            pltpu.async_copy(wt.at[pl.ds(h0, 8), pl.ds(c * C, C)],
                             w_v.at[slot], sem)

        def wait(slot, c, wt=wt_hbm):
            sem = sems[slot]
            pltpu.make_async_copy(src_hbm.at[pl.ds(c * C, C)],
                                  src_v.at[slot], sem).wait()
            pltpu.make_async_copy(tgt_hbm.at[pl.ds(c * C, C)],
                                  tgt_v.at[slot], sem).wait()
            pltpu.make_async_copy(wt.at[pl.ds(h0, 8), pl.ds(c * C, C)],
                                  w_v.at[slot], sem).wait()

        def compute(slot):
            @plsc.parallel_loop(0, C // 16, unroll=4)
            def _(i):
                s16 = src_v[slot, pl.ds(i * 16, 16)]
                t16 = tgt_v[slot, pl.ds(i * 16, 16)]
                g16 = plsc.load_gather(cal_v, [s16])
                for j in range(8):
                    w16 = w_v[slot, j, pl.ds(i * 16, 16)]
                    idx = t16 + (j * N) if j else t16
                    plsc.addupdate_scatter(acc_v, [idx], g16 * w16)

        start(0, 0)

        @plsc.parallel_loop(0, (8 * N) // 16, unroll=8)
        def _(i):
            acc_v[pl.ds(i * 16, 16)] = zero16

        @pl.loop(0, NCHUNK, step=2)
        def _(c):
            start(1, c + 1)
            wait(0, c)
            compute(0)

            @pl.when(c + 2 < NCHUNK)
            def _():
                start(0, c + 2)

            wait(1, c + 1)
            compute(1)

        for j in range(8):
            pltpu.sync_copy(acc_v.at[pl.ds(j * N, N)], out_hbm.at[g, h0 + j, sid])


def _sc_sparse(calcium_t, src, tgt, wt_z, wt_r, wt_h):
    mesh = plsc.VectorSubcoreMesh(core_axis_name="c", subcore_axis_name="s")
    fn = pl.kernel(
        _sc_body,
        out_type=jax.ShapeDtypeStruct((3, H, B, N), jnp.float32),
        mesh=mesh,
        scratch_types=[
            pltpu.VMEM((N,), jnp.float32),
            pltpu.VMEM((2, C), jnp.int32),
            pltpu.VMEM((2, C), jnp.int32),
            pltpu.VMEM((2, 8, C), jnp.float32),
            pltpu.VMEM((8 * N,), jnp.float32),
            pltpu.SemaphoreType.DMA,
            pltpu.SemaphoreType.DMA,
        ],
        compiler_params=pltpu.CompilerParams(
            needs_layout_passes=False, use_tc_tiling_on_sc=False),
    )
    return fn(calcium_t, src, tgt, wt_z, wt_r, wt_h)


def _rec_body(hid_ref, uz_ref, ur_ref, bz_ref, br_ref, rz_ref, rr_ref):
    hid = hid_ref[...]
    uz = uz_ref[...]
    ur = ur_ref[...]
    rec_z = jnp.broadcast_to(bz_ref[...][:, None, :], (H, B, NB))
    rec_r = jnp.broadcast_to(br_ref[...][:, None, :], (H, B, NB))
    for h in range(H):
        hh = hid[h][None, :, :]
        rec_z = rec_z + hh * uz[h][:, None, :]
        rec_r = rec_r + hh * ur[h][:, None, :]
    rz_ref[...] = rec_z
    rr_ref[...] = rec_r


def _gate_body(hid_ref, uh_ref, iz_ref, ir_ref, ih_ref, rz_ref, rr_ref,
               bh_ref, w_ref, hnew_ref, cal_ref):
    hid = hid_ref[...]
    uh = uh_ref[...]
    z = jax.nn.sigmoid(iz_ref[...] + rz_ref[...])
    r = jax.nn.sigmoid(ir_ref[...] + rr_ref[...])

    rh = r * hid
    rec_h = jnp.zeros((H, B, NB), jnp.float32)
    for h in range(H):
        rec_h = rec_h + rh[h][None, :, :] * uh[h][:, None, :]
    h_t = jnp.tanh(ih_ref[...] + rec_h + bh_ref[...][:, None, :])

    hnew = (1.0 - z) * hid + z * h_t
    hnew_ref[...] = hnew

    w = w_ref[...][:, 0]
    cal_ref[...] = jax.nn.relu(jnp.sum(hnew * w[:, None, None], axis=0))


_HBN = pl.BlockSpec((H, B, NB), lambda j: (0, 0, j))
_HN = pl.BlockSpec((H, NB), lambda j: (0, j))


def _rec_call(hid_t, u_z, u_r, b_z, b_r):
    return pl.pallas_call(
        _rec_body,
        grid=(pl.cdiv(N, NB),),
        in_specs=[_HBN, _HBN, _HBN, _HN, _HN],
        out_specs=[_HBN, _HBN],
        out_shape=[
            jax.ShapeDtypeStruct((H, B, N), jnp.float32),
            jax.ShapeDtypeStruct((H, B, N), jnp.float32),
        ],
    )(hid_t, u_z, u_r, b_z, b_r)


def _gate_call(hid_t, u_h, i_z, i_r, i_h, r_z, r_r, b_h, w2d):
    wspec = pl.BlockSpec((H, 128), lambda j: (0, 0))
    return pl.pallas_call(
        _gate_body,
        grid=(pl.cdiv(N, NB),),
        in_specs=[_HBN, _HBN, _HBN, _HBN, _HBN, _HBN, _HBN, _HN, wspec],
        out_specs=[_HBN, pl.BlockSpec((B, NB), lambda j: (0, j))],
        out_shape=[
            jax.ShapeDtypeStruct((H, B, N), jnp.float32),
            jax.ShapeDtypeStruct((B, N), jnp.float32),
        ],
    )(hid_t, u_h, i_z, i_r, i_h, r_z, r_r, b_h, w2d)


def kernel(calcium_t, hidden, W_z_values, W_r_values, W_h_values,
           U_z, U_r, U_h, b_z, b_r, b_h, output_projection, src, tgt):
    hid_t = jnp.transpose(hidden, (2, 0, 1))
    u_z = jnp.transpose(U_z, (1, 2, 0))
    u_r = jnp.transpose(U_r, (1, 2, 0))
    u_h = jnp.transpose(U_h, (1, 2, 0))
    bz_t = b_z.T
    br_t = b_r.T
    bh_t = b_h.T
    w2d = jnp.broadcast_to(output_projection[:, None], (H, 128))
    wt_z = W_z_values.T
    wt_r = W_r_values.T
    wt_h = W_h_values.T

    inp = _sc_sparse(calcium_t, src, tgt, wt_z, wt_r, wt_h)

    r_z, r_r = _rec_call(hid_t, u_z, u_r, bz_t, br_t)

    hnew_t, calcium_t1 = _gate_call(hid_t, u_h, inp[0], inp[1], inp[2],
                                    r_z, r_r, bh_t, w2d)
    hidden_new = jnp.transpose(hnew_t, (1, 2, 0))
    return (calcium_t1, hidden_new)
